# baseline (speedup 1.0000x reference)
import functools

import jax
import jax.numpy as jnp
from jax import lax
from jax.experimental import pallas as pl
from jax.experimental.pallas import tpu as pltpu
from jax.experimental.pallas import tpu_sc as plsc

EMB = 64
NC, NS = 2, 16
NW = NC * NS


def _sc_gather4(user_ids, artist_ids, gmf_user, gmf_artist, mlp_user, mlp_artist):
    B = user_ids.shape[0]
    b_per_w = B // NW
    mesh = plsc.VectorSubcoreMesh(core_axis_name="c", subcore_axis_name="s")

    @functools.partial(
        pl.kernel,
        out_type=[jax.ShapeDtypeStruct((B, EMB), jnp.float32)] * 4,
        mesh=mesh,
        scratch_types=[
            pltpu.VMEM((b_per_w,), jnp.int32),
            pltpu.VMEM((b_per_w,), jnp.int32),
            pltpu.VMEM((b_per_w, EMB), jnp.float32),
            pltpu.VMEM((b_per_w, EMB), jnp.float32),
            pltpu.SemaphoreType.DMA,
            pltpu.SemaphoreType.DMA,
        ],
        compiler_params=pltpu.CompilerParams(use_tc_tiling_on_sc=False),
    )
    def gather_kernel(uid, aid, gu, ga, mu, ma, o_gu, o_ga, o_mu, o_ma,
                      idx_u, idx_a, rows0, rows1, sem0, sem1):
        wid = lax.axis_index("s") * NC + lax.axis_index("c")
        base = wid * b_per_w
        pltpu.sync_copy(uid.at[pl.ds(base, b_per_w)], idx_u)
        pltpu.sync_copy(aid.at[pl.ds(base, b_per_w)], idx_a)
        c0 = pltpu.async_copy(gu.at[idx_u], rows0, sem0)
        c1 = pltpu.async_copy(ga.at[idx_a], rows1, sem1)
        c0.wait()
        pltpu.sync_copy(rows0, o_gu.at[pl.ds(base, b_per_w)])
        c1.wait()
        pltpu.sync_copy(rows1, o_ga.at[pl.ds(base, b_per_w)])
        c0 = pltpu.async_copy(mu.at[idx_u], rows0, sem0)
        c1 = pltpu.async_copy(ma.at[idx_a], rows1, sem1)
        c0.wait()
        pltpu.sync_copy(rows0, o_mu.at[pl.ds(base, b_per_w)])
        c1.wait()
        pltpu.sync_copy(rows1, o_ma.at[pl.ds(base, b_per_w)])

    return gather_kernel(user_ids, artist_ids, gmf_user, gmf_artist,
                         mlp_user, mlp_artist)


def _tc_mlp(gmf_u, gmf_a, mlp_u, mlp_a, W1, b1, W2, b2, W3, b3, Wf, bf):
    B = gmf_u.shape[0]
    BB = 2048
    w1u = W1[:, :EMB].T
    w1a = W1[:, EMB:].T
    w2t = W2.T
    w3t = W3.T
    wfg = Wf[:, :EMB]
    wfh = Wf[:, EMB:]
    b1r = b1.reshape(1, -1)
    b2r = b2.reshape(1, -1)
    b3r = b3.reshape(1, -1)
    bfr = bf.reshape(1, 1)

    def body(gu, ga, mu, ma, w1u_r, w1a_r, w2_r, w3_r, wfg_r, wfh_r,
             b1_r, b2_r, b3_r, bf_r, out_r):
        dot = functools.partial(jnp.dot, preferred_element_type=jnp.float32)
        h = jnp.maximum(dot(mu[...], w1u_r[...]) + dot(ma[...], w1a_r[...])
                        + b1_r[...], 0.0)
        h = jnp.maximum(dot(h, w2_r[...]) + b2_r[...], 0.0)
        h = jnp.maximum(dot(h, w3_r[...]) + b3_r[...], 0.0)
        g = jnp.sum(gu[...] * ga[...] * wfg_r[...], axis=1, keepdims=True)
        m = jnp.sum(h * wfh_r[...], axis=1, keepdims=True)
        out_r[...] = jax.nn.sigmoid(g + m + bf_r[...])

    full = lambda a: pl.BlockSpec(a.shape, lambda i: (0, 0))
    blk = pl.BlockSpec((BB, EMB), lambda i: (i, 0))
    out = pl.pallas_call(
        body,
        grid=(B // BB,),
        in_specs=[blk, blk, blk, blk,
                  full(w1u), full(w1a), full(w2t), full(w3t),
                  full(wfg), full(wfh), full(b1r), full(b2r), full(b3r),
                  full(bfr)],
        out_specs=pl.BlockSpec((BB, 1), lambda i: (i, 0)),
        out_shape=jax.ShapeDtypeStruct((B, 1), jnp.float32),
    )(gmf_u, gmf_a, mlp_u, mlp_a, w1u, w1a, w2t, w3t, wfg, wfh,
      b1r, b2r, b3r, bfr)
    return out[:, 0]


def kernel(user_ids, artist_ids, gmf_user, gmf_artist, mlp_user, mlp_artist,
           W1, b1, W2, b2, W3, b3, Wf, bf):
    gu, ga, mu, ma = _sc_gather4(user_ids, artist_ids, gmf_user, gmf_artist,
                                 mlp_user, mlp_artist)
    return _tc_mlp(gu, ga, mu, ma, W1, b1, W2, b2, W3, b3, Wf, bf)


# speedup vs baseline: 1.5304x; 1.5304x over previous
import functools

import jax
import jax.numpy as jnp
from jax import lax
from jax.experimental import pallas as pl
from jax.experimental.pallas import tpu as pltpu
from jax.experimental.pallas import tpu_sc as plsc

EMB = 64
NC, NS = 2, 16
NW = NC * NS


def _sc_gather4(user_ids, artist_ids, gmf_user, gmf_artist, mlp_user, mlp_artist):
    B = user_ids.shape[0]
    b_per_w = B // NW
    CH = b_per_w // 2
    mesh = plsc.VectorSubcoreMesh(core_axis_name="c", subcore_axis_name="s")

    @functools.partial(
        pl.kernel,
        out_type=[jax.ShapeDtypeStruct((B, EMB), jnp.float32)] * 4,
        mesh=mesh,
        scratch_types=[
            pltpu.VMEM((b_per_w,), jnp.int32),
            pltpu.VMEM((b_per_w,), jnp.int32),
            pltpu.VMEM((CH, EMB), jnp.float32),
            pltpu.VMEM((CH, EMB), jnp.float32),
            pltpu.SemaphoreType.DMA,
            pltpu.SemaphoreType.DMA,
        ],
    )
    def gather_kernel(uid, aid, gu, ga, mu, ma, o_gu, o_ga, o_mu, o_ma,
                      idx_u, idx_a, rows0, rows1, sem0, sem1):
        wid = lax.axis_index("s") * NC + lax.axis_index("c")
        base = wid * b_per_w
        L = 16

        def fire(task, buf):
            table, idx_v, _, chunk = task
            rows, sem = buf

            def lbody(g, carry):
                vec = idx_v[pl.ds(chunk * CH + g * L, L)]
                for k in range(L):
                    pltpu.async_copy(table.at[pl.ds(vec[k], 1)],
                                     rows.at[pl.ds(g * L + k, 1)], sem)
                return carry
            lax.fori_loop(0, CH // L, lbody, 0)

        def finish(task, buf):
            table, _, out, chunk = task
            rows, sem = buf
            pltpu.make_async_copy(table.at[pl.ds(0, CH)], rows, sem).wait()
            pltpu.sync_copy(rows, out.at[pl.ds(base + chunk * CH, CH)])

        pltpu.sync_copy(uid.at[pl.ds(base, b_per_w)], idx_u)
        pltpu.sync_copy(aid.at[pl.ds(base, b_per_w)], idx_a)

        tasks = [(t, iv, o, c)
                 for (t, iv, o) in ((gu, idx_u, o_gu), (ga, idx_a, o_ga),
                                    (mu, idx_u, o_mu), (ma, idx_a, o_ma))
                 for c in (0, 1)]
        bufs = [(rows0, sem0), (rows1, sem1)]
        for k, task in enumerate(tasks):
            if k >= 2:
                finish(tasks[k - 2], bufs[k % 2])
            fire(task, bufs[k % 2])
        finish(tasks[-2], bufs[0])
        finish(tasks[-1], bufs[1])

    return gather_kernel(user_ids, artist_ids, gmf_user, gmf_artist,
                         mlp_user, mlp_artist)


def _tc_mlp(gmf_u, gmf_a, mlp_u, mlp_a, W1, b1, W2, b2, W3, b3, Wf, bf):
    B = gmf_u.shape[0]
    BB = 2048
    w1u = W1[:, :EMB].T
    w1a = W1[:, EMB:].T
    w2t = W2.T
    w3t = W3.T
    wfg = Wf[:, :EMB]
    wfh = Wf[:, EMB:]
    b1r = b1.reshape(1, -1)
    b2r = b2.reshape(1, -1)
    b3r = b3.reshape(1, -1)
    bfr = bf.reshape(1, 1)

    def body(gu, ga, mu, ma, w1u_r, w1a_r, w2_r, w3_r, wfg_r, wfh_r,
             b1_r, b2_r, b3_r, bf_r, out_r):
        dot = functools.partial(jnp.dot, preferred_element_type=jnp.float32)
        h = jnp.maximum(dot(mu[...], w1u_r[...]) + dot(ma[...], w1a_r[...])
                        + b1_r[...], 0.0)
        h = jnp.maximum(dot(h, w2_r[...]) + b2_r[...], 0.0)
        h = jnp.maximum(dot(h, w3_r[...]) + b3_r[...], 0.0)
        g = jnp.sum(gu[...] * ga[...] * wfg_r[...], axis=1, keepdims=True)
        m = jnp.sum(h * wfh_r[...], axis=1, keepdims=True)
        out_r[...] = jax.nn.sigmoid(g + m + bf_r[...])

    full = lambda a: pl.BlockSpec(a.shape, lambda i: (0, 0))
    blk = pl.BlockSpec((BB, EMB), lambda i: (i, 0))
    out = pl.pallas_call(
        body,
        grid=(B // BB,),
        in_specs=[blk, blk, blk, blk,
                  full(w1u), full(w1a), full(w2t), full(w3t),
                  full(wfg), full(wfh), full(b1r), full(b2r), full(b3r),
                  full(bfr)],
        out_specs=pl.BlockSpec((BB, 1), lambda i: (i, 0)),
        out_shape=jax.ShapeDtypeStruct((B, 1), jnp.float32),
    )(gmf_u, gmf_a, mlp_u, mlp_a, w1u, w1a, w2t, w3t, wfg, wfh,
      b1r, b2r, b3r, bfr)
    return out[:, 0]


def kernel(user_ids, artist_ids, gmf_user, gmf_artist, mlp_user, mlp_artist,
           W1, b1, W2, b2, W3, b3, Wf, bf):
    gu, ga, mu, ma = _sc_gather4(user_ids, artist_ids, gmf_user, gmf_artist,
                                 mlp_user, mlp_artist)
    return _tc_mlp(gu, ga, mu, ma, W1, b1, W2, b2, W3, b3, Wf, bf)
